# baseline (speedup 1.0000x reference)
import jax
import jax.numpy as jnp
from jax import lax
from jax.experimental import pallas as pl
from jax.experimental.pallas import tpu as pltpu
from jax.experimental.pallas import tpu_sc as plsc

N = 10000
NPAD = 10240
OWN = NPAD // 2
TRASH = OWN
E = 160000
EPS = E // 16
BLK_ACC = 96
EPSP = 10080
BLK_TGT = 80
D = 256
H = 256
HH = 128
NCLS = 64
B = 1024

_SC_MESH = dict(core_axis_name="c", subcore_axis_name="s")
_SC_PARAMS = pltpu.CompilerParams(needs_layout_passes=False)


HSIZE = 5376


def _make_sc_degree():
    mesh = plsc.VectorSubcoreMesh(**_SC_MESH)

    def body(dst_hbm, deg_hbm, dstb, hist, rbuf, sbuf, sh, sem):
        c = lax.axis_index("c")
        s = lax.axis_index("s")

        def zero_chunk(i, _):
            hist[pl.ds(i * 16, 16)] = jnp.zeros((16,), jnp.float32)
            return _
        lax.fori_loop(0, HSIZE // 16, zero_chunk, None)

        pltpu.sync_copy(dst_hbm.at[pl.ds(s * EPS, EPS)], dstb)

        cbase = c * OWN

        def count(i, _):
            d = dstb[pl.ds(i * 16, 16)]
            dl = d - cbase
            ok = (dl >= 0) & (dl < OWN)
            idx = jnp.where(ok, dl, TRASH)
            plsc.addupdate_scatter(hist, [idx], jnp.ones((16,), jnp.float32))
            return _
        lax.fori_loop(0, EPS // 16, count, None)

        pltpu.sync_copy(hist, sh.at[s])
        plsc.subcore_barrier()

        @pl.when(s < 8)
        def _():
            for k in range(16):
                pltpu.sync_copy(sh.at[k].at[pl.ds(s * 640, 640)], rbuf.at[k])

            def red_chunk(i, _):
                acc = rbuf[0, pl.ds(i * 16, 16)]
                for k in range(1, 16):
                    acc = acc + rbuf[k, pl.ds(i * 16, 16)]
                sbuf[pl.ds(i * 16, 16)] = acc
                return _
            lax.fori_loop(0, 40, red_chunk, None)

            pltpu.sync_copy(sbuf, deg_hbm.at[pl.ds(cbase + s * 640, 640)])

    return pl.kernel(
        body,
        out_type=jax.ShapeDtypeStruct((NPAD,), jnp.float32),
        mesh=mesh,
        scratch_types=[
            pltpu.VMEM((EPS,), jnp.int32),
            pltpu.VMEM((HSIZE,), jnp.float32),
            pltpu.VMEM((16, 640), jnp.float32),
            pltpu.VMEM((640,), jnp.float32),
            pltpu.VMEM_SHARED((16, HSIZE), jnp.float32),
            pltpu.SemaphoreType.DMA,
        ],
        compiler_params=_SC_PARAMS,
    )


def _make_sc_spmm(tail):
    mesh = plsc.VectorSubcoreMesh(**_SC_MESH)

    NS = 2
    blk = BLK_ACC if tail == "acc" else BLK_TGT
    eps = EPSP if tail == "acc" else EPS
    nblk = eps // blk

    def spmm_body(z_hbm, src_hbm, dst_hbm, srcb, dstb, srcw, idxw, gbuf,
                  acc_sh, gsem, ssem):
        c = lax.axis_index("c")
        s = lax.axis_index("s")
        zc = z_hbm.at[c]

        pltpu.sync_copy(zc.at[pl.ds(s * 640, 640)],
                        acc_sh.at[pl.ds(s * 640, 640)])

        pltpu.sync_copy(src_hbm.at[pl.ds(s * eps, eps)], srcb)
        pltpu.sync_copy(dst_hbm.at[pl.ds(s * eps, eps)], dstb)

        plsc.subcore_barrier()

        def stage_and_fire(b, k):
            base = b * blk
            for j in range(blk // 16):
                idxw[k, pl.ds(j * 16, 16)] = dstb[pl.ds(base + j * 16, 16)]
                srcw[k, pl.ds(j * 16, 16)] = srcb[pl.ds(base + j * 16, 16)]
            pltpu.async_copy(zc.at[srcw.at[k]], gbuf.at[k], gsem[k])

        def finish(k):
            pltpu.make_async_copy(zc.at[srcw.at[k]], gbuf.at[k],
                                  gsem[k]).wait()
            pltpu.async_copy(gbuf.at[k], acc_sh.at[idxw.at[k]], ssem[k],
                             add=True)

        def drain(k):
            pltpu.make_async_copy(gbuf.at[k], acc_sh.at[idxw.at[k]],
                                  ssem[k]).wait()

        stage_and_fire(0, 0)

        def group(g, _):
            for k in range(NS):
                b = g * NS + k
                nk = (k + 1) % NS
                if k == 0:
                    @pl.when(g > 0)
                    def _():
                        drain(nk)
                else:
                    drain(nk)
                stage_and_fire(b + 1, nk)
                finish(k)
            return _
        lax.fori_loop(0, (nblk - 1) // NS, group, None)
        finish(0)
        drain(1)
        drain(0)

        plsc.subcore_barrier()
        return c, s

    common_scratch = [
        pltpu.VMEM((eps,), jnp.int32),
        pltpu.VMEM((eps,), jnp.int32),
        pltpu.VMEM((NS, blk), jnp.int32),
        pltpu.VMEM((NS, blk), jnp.int32),
        pltpu.VMEM((NS, blk, HH), jnp.float32),
        pltpu.VMEM_SHARED((NPAD, HH), jnp.float32),
        [pltpu.SemaphoreType.DMA] * NS,
        [pltpu.SemaphoreType.DMA] * NS,
    ]

    if tail == "acc":
        def body(z_hbm, src_hbm, dst_hbm, acc_hbm, srcb, dstb, srcw, idxw,
                 gbuf, acc_sh, gsem, ssem):
            c, s = spmm_body(z_hbm, src_hbm, dst_hbm, srcb, dstb, srcw, idxw,
                             gbuf, acc_sh, gsem, ssem)
            pltpu.sync_copy(acc_sh.at[pl.ds(s * 640, 640)],
                            acc_hbm.at[c].at[pl.ds(s * 640, 640)])

        return pl.kernel(
            body,
            out_type=jax.ShapeDtypeStruct((2, NPAD, HH), jnp.float32),
            mesh=mesh,
            scratch_types=common_scratch,
            compiler_params=_SC_PARAMS,
        )

    TPS = B // 16

    def body(z_hbm, src_hbm, dst_hbm, tgt_hbm, t3_hbm,
             srcb, dstb, srcw, idxw, gbuf, acc_sh, gsem, ssem, tbuf, trows):
        c, s = spmm_body(z_hbm, src_hbm, dst_hbm, srcb, dstb, srcw, idxw,
                         gbuf, acc_sh, gsem, ssem)
        base = s * TPS
        pltpu.sync_copy(tgt_hbm.at[pl.ds(base, TPS)], tbuf)
        def tgt_round(h, _):
            off = h * (TPS // 4)
            tb = tbuf.at[pl.ds(off, TPS // 4)]
            pltpu.async_copy(acc_sh.at[tb], trows, gsem[0]).wait()
            pltpu.sync_copy(
                trows, t3_hbm.at[c].at[pl.ds(base + off, TPS // 4)])
            return _
        lax.fori_loop(0, 4, tgt_round, None)

    return pl.kernel(
        body,
        out_type=jax.ShapeDtypeStruct((2, B, HH), jnp.float32),
        mesh=mesh,
        scratch_types=common_scratch + [
            pltpu.VMEM((TPS,), jnp.int32),
            pltpu.VMEM((TPS // 4, HH), jnp.float32),
        ],
        compiler_params=_SC_PARAMS,
    )


_RB = 256


def _split_store(z3_ref, z):
    z3_ref[0] = z[:, :HH]
    z3_ref[1] = z[:, HH:]


def _tc_in_body(deg_ref, x_ref, w_ref, z3_ref):
    dinv = lax.rsqrt(deg_ref[...] + 1.0)
    z = jnp.dot(x_ref[...], w_ref[...],
                preferred_element_type=jnp.float32) * dinv
    _split_store(z3_ref, z)


def _make_tc_in():
    grid = (NPAD // _RB,)
    return pl.pallas_call(
        _tc_in_body,
        grid=grid,
        in_specs=[
            pl.BlockSpec((_RB, 1), lambda i: (i, 0)),
            pl.BlockSpec((_RB, D), lambda i: (i, 0)),
            pl.BlockSpec((D, H), lambda i: (0, 0)),
        ],
        out_specs=pl.BlockSpec((2, _RB, HH), lambda i: (0, i, 0)),
        out_shape=jax.ShapeDtypeStruct((2, NPAD, HH), jnp.float32),
    )


def _tc_mid_body(acc3_ref, deg_ref, b_ref, w_ref, z3_ref):
    acc = jnp.concatenate([acc3_ref[0], acc3_ref[1]], axis=1)
    dinv = lax.rsqrt(deg_ref[...] + 1.0)
    h = jnp.maximum(acc * dinv + b_ref[...], 0.0)
    z = jnp.dot(h, w_ref[...], preferred_element_type=jnp.float32) * dinv
    _split_store(z3_ref, z)


def _make_tc_mid():
    grid = (NPAD // _RB,)
    return pl.pallas_call(
        _tc_mid_body,
        grid=grid,
        in_specs=[
            pl.BlockSpec((2, _RB, HH), lambda i: (0, i, 0)),
            pl.BlockSpec((_RB, 1), lambda i: (i, 0)),
            pl.BlockSpec((1, H), lambda i: (0, 0)),
            pl.BlockSpec((H, H), lambda i: (0, 0)),
        ],
        out_specs=pl.BlockSpec((2, _RB, HH), lambda i: (0, i, 0)),
        out_shape=jax.ShapeDtypeStruct((2, NPAD, HH), jnp.float32),
    )


def _gates(g):
    i = jax.nn.sigmoid(g[:, 0:H])
    gg = jnp.tanh(g[:, 2 * H:3 * H])
    o = jax.nn.sigmoid(g[:, 3 * H:4 * H])
    return o * jnp.tanh(i * gg)


def _tc_head_body(t3_ref, deg_ref, ti_ref, b2_ref, wih0_ref, b0_ref,
                  wih1_ref, b1_ref, fcw_ref, fcb_ref, o_ref):
    dn = (((1,), (1,)), ((), ()))
    traw = jnp.concatenate([t3_ref[0], t3_ref[1]], axis=1)
    tid = ti_ref[...]
    KC = 2048

    def chunk(k, acc):
        kb = k * KC
        ids = kb + lax.broadcasted_iota(jnp.int32, (1, KC), 1)
        oh = (tid == ids).astype(jnp.float32)
        dsl = deg_ref[pl.ds(kb, KC), 0:1]
        return acc + jnp.dot(oh, dsl, preferred_element_type=jnp.float32)

    degt = lax.fori_loop(0, NPAD // KC, chunk,
                         jnp.zeros((_RB, 1), jnp.float32))
    dinv_t = lax.rsqrt(degt + 1.0)
    t = jnp.maximum(traw * dinv_t + b2_ref[...], 0.0)
    g0 = lax.dot_general(t, wih0_ref[...], dn,
                         preferred_element_type=jnp.float32) + b0_ref[...]
    h1 = _gates(g0)
    g1 = lax.dot_general(h1, wih1_ref[...], dn,
                         preferred_element_type=jnp.float32) + b1_ref[...]
    h2 = _gates(g1)
    o_ref[...] = lax.dot_general(h2, fcw_ref[...], dn,
                                 preferred_element_type=jnp.float32) + fcb_ref[...]


def _make_tc_head():
    grid = (B // _RB,)
    return pl.pallas_call(
        _tc_head_body,
        grid=grid,
        in_specs=[
            pl.BlockSpec((2, _RB, HH), lambda i: (0, i, 0)),
            pl.BlockSpec((NPAD, 1), lambda i: (0, 0)),
            pl.BlockSpec((_RB, 1), lambda i: (i, 0)),
            pl.BlockSpec((1, H), lambda i: (0, 0)),
            pl.BlockSpec((4 * H, H), lambda i: (0, 0)),
            pl.BlockSpec((1, 4 * H), lambda i: (0, 0)),
            pl.BlockSpec((4 * H, H), lambda i: (0, 0)),
            pl.BlockSpec((1, 4 * H), lambda i: (0, 0)),
            pl.BlockSpec((NCLS, H), lambda i: (0, 0)),
            pl.BlockSpec((1, NCLS), lambda i: (0, 0)),
        ],
        out_specs=pl.BlockSpec((_RB, NCLS), lambda i: (i, 0)),
        out_shape=jax.ShapeDtypeStruct((B, NCLS), jnp.float32),
    )


def kernel(x, edge_index, target_node_index, W1, b1, W2, b2, Wih0, Whh0,
           bih0, bhh0, Wih1, Whh1, bih1, bhh1, fcW, fcb):
    src = edge_index[0]
    dst = edge_index[1]
    ppe = EPSP - EPS
    padv = (N + jnp.arange(16 * ppe).reshape(16, ppe) % (NPAD - N)
            ).astype(src.dtype)
    srcp = jnp.concatenate([src.reshape(16, EPS), padv], axis=1).reshape(-1)
    dstp = jnp.concatenate([dst.reshape(16, EPS), padv], axis=1).reshape(-1)
    x_pad = jnp.pad(x, ((0, NPAD - N), (0, 0)))

    deg = _make_sc_degree()(dst).reshape(NPAD, 1)

    z3 = _make_tc_in()(deg, x_pad, W1)
    acc3 = _make_sc_spmm("acc")(z3, srcp, dstp)
    z3 = _make_tc_mid()(acc3, deg, b1.reshape(1, H), W2)
    t3 = _make_sc_spmm("tgt")(z3, src, dst, target_node_index)
    return _make_tc_head()(t3, deg,
                           target_node_index.reshape(B, 1),
                           b2.reshape(1, H),
                           Wih0, (bih0 + bhh0).reshape(1, 4 * H),
                           Wih1, (bih1 + bhh1).reshape(1, 4 * H),
                           fcW, fcb.reshape(1, NCLS))


# speedup vs baseline: 1.0198x; 1.0198x over previous
import jax
import jax.numpy as jnp
from jax import lax
from jax.experimental import pallas as pl
from jax.experimental.pallas import tpu as pltpu
from jax.experimental.pallas import tpu_sc as plsc

N = 10000
NPAD = 10240
OWN = NPAD // 2
TRASH = OWN
E = 160000
EPS = E // 16
BLK_ACC = 96
EPSP = 10080
BLK_TGT = 80
D = 256
H = 256
HH = 128
NCLS = 64
B = 1024

_SC_MESH = dict(core_axis_name="c", subcore_axis_name="s")
_SC_PARAMS = pltpu.CompilerParams(needs_layout_passes=False)


HSIZE = 5376


def _make_sc_degree():
    mesh = plsc.VectorSubcoreMesh(**_SC_MESH)

    def body(dst_hbm, deg_hbm, dstb, hist, rbuf, sbuf, sh, sem):
        c = lax.axis_index("c")
        s = lax.axis_index("s")

        def zero_chunk(i, _):
            hist[pl.ds(i * 16, 16)] = jnp.zeros((16,), jnp.float32)
            return _
        lax.fori_loop(0, HSIZE // 16, zero_chunk, None)

        pltpu.sync_copy(dst_hbm.at[pl.ds(s * EPS, EPS)], dstb)

        cbase = c * OWN

        def count(i, _):
            d = dstb[pl.ds(i * 16, 16)]
            dl = d - cbase
            ok = (dl >= 0) & (dl < OWN)
            idx = jnp.where(ok, dl, TRASH)
            plsc.addupdate_scatter(hist, [idx], jnp.ones((16,), jnp.float32))
            return _
        lax.fori_loop(0, EPS // 16, count, None)

        pltpu.sync_copy(hist, sh.at[s])
        plsc.subcore_barrier()

        @pl.when(s < 8)
        def _():
            for k in range(16):
                pltpu.sync_copy(sh.at[k].at[pl.ds(s * 640, 640)], rbuf.at[k])

            def red_chunk(i, _):
                acc = rbuf[0, pl.ds(i * 16, 16)]
                for k in range(1, 16):
                    acc = acc + rbuf[k, pl.ds(i * 16, 16)]
                sbuf[pl.ds(i * 16, 16)] = acc
                return _
            lax.fori_loop(0, 40, red_chunk, None)

            pltpu.sync_copy(sbuf, deg_hbm.at[pl.ds(cbase + s * 640, 640)])

    return pl.kernel(
        body,
        out_type=jax.ShapeDtypeStruct((NPAD,), jnp.float32),
        mesh=mesh,
        scratch_types=[
            pltpu.VMEM((EPS,), jnp.int32),
            pltpu.VMEM((HSIZE,), jnp.float32),
            pltpu.VMEM((16, 640), jnp.float32),
            pltpu.VMEM((640,), jnp.float32),
            pltpu.VMEM_SHARED((16, HSIZE), jnp.float32),
            pltpu.SemaphoreType.DMA,
        ],
        compiler_params=_SC_PARAMS,
    )


def _make_sc_spmm(tail):
    mesh = plsc.VectorSubcoreMesh(**_SC_MESH)

    NS = 2
    blk = BLK_ACC
    eps = EPSP
    nblk = eps // blk

    def spmm_body(z_hbm, src_hbm, dst_hbm, srcb, dstb, srcw, idxw, gbuf,
                  acc_sh, gsem, ssem):
        c = lax.axis_index("c")
        s = lax.axis_index("s")
        zc = z_hbm.at[c]

        pltpu.sync_copy(zc.at[pl.ds(s * 640, 640)],
                        acc_sh.at[pl.ds(s * 640, 640)])

        pltpu.sync_copy(src_hbm.at[pl.ds(s * eps, eps)], srcb)
        pltpu.sync_copy(dst_hbm.at[pl.ds(s * eps, eps)], dstb)

        plsc.subcore_barrier()

        def stage_and_fire(b, k):
            base = b * blk
            for j in range(blk // 16):
                idxw[k, pl.ds(j * 16, 16)] = dstb[pl.ds(base + j * 16, 16)]
                srcw[k, pl.ds(j * 16, 16)] = srcb[pl.ds(base + j * 16, 16)]
            pltpu.async_copy(zc.at[srcw.at[k]], gbuf.at[k], gsem[k])

        def finish(k):
            pltpu.make_async_copy(zc.at[srcw.at[k]], gbuf.at[k],
                                  gsem[k]).wait()
            pltpu.async_copy(gbuf.at[k], acc_sh.at[idxw.at[k]], ssem[k],
                             add=True)

        def drain(k):
            pltpu.make_async_copy(gbuf.at[k], acc_sh.at[idxw.at[k]],
                                  ssem[k]).wait()

        stage_and_fire(0, 0)

        def group(g, _):
            for k in range(NS):
                b = g * NS + k
                nk = (k + 1) % NS
                if k == 0:
                    @pl.when(g > 0)
                    def _():
                        drain(nk)
                else:
                    drain(nk)
                stage_and_fire(b + 1, nk)
                finish(k)
            return _
        lax.fori_loop(0, (nblk - 1) // NS, group, None)
        finish(0)
        drain(1)
        drain(0)

        plsc.subcore_barrier()
        return c, s

    common_scratch = [
        pltpu.VMEM((eps,), jnp.int32),
        pltpu.VMEM((eps,), jnp.int32),
        pltpu.VMEM((NS, blk), jnp.int32),
        pltpu.VMEM((NS, blk), jnp.int32),
        pltpu.VMEM((NS, blk, HH), jnp.float32),
        pltpu.VMEM_SHARED((NPAD, HH), jnp.float32),
        [pltpu.SemaphoreType.DMA] * NS,
        [pltpu.SemaphoreType.DMA] * NS,
    ]

    if tail == "acc":
        def body(z_hbm, src_hbm, dst_hbm, acc_hbm, srcb, dstb, srcw, idxw,
                 gbuf, acc_sh, gsem, ssem):
            c, s = spmm_body(z_hbm, src_hbm, dst_hbm, srcb, dstb, srcw, idxw,
                             gbuf, acc_sh, gsem, ssem)
            pltpu.sync_copy(acc_sh.at[pl.ds(s * 640, 640)],
                            acc_hbm.at[c].at[pl.ds(s * 640, 640)])

        return pl.kernel(
            body,
            out_type=jax.ShapeDtypeStruct((2, NPAD, HH), jnp.float32),
            mesh=mesh,
            scratch_types=common_scratch,
            compiler_params=_SC_PARAMS,
        )

    TPS = B // 16

    def body(z_hbm, src_hbm, dst_hbm, tgt_hbm, t3_hbm,
             srcb, dstb, srcw, idxw, gbuf, acc_sh, gsem, ssem, tbuf, trows):
        c, s = spmm_body(z_hbm, src_hbm, dst_hbm, srcb, dstb, srcw, idxw,
                         gbuf, acc_sh, gsem, ssem)
        base = s * TPS
        pltpu.sync_copy(tgt_hbm.at[pl.ds(base, TPS)], tbuf)
        def tgt_round(h, _):
            off = h * (TPS // 4)
            tb = tbuf.at[pl.ds(off, TPS // 4)]
            pltpu.async_copy(acc_sh.at[tb], trows, gsem[0]).wait()
            pltpu.sync_copy(
                trows, t3_hbm.at[c].at[pl.ds(base + off, TPS // 4)])
            return _
        lax.fori_loop(0, 4, tgt_round, None)

    return pl.kernel(
        body,
        out_type=jax.ShapeDtypeStruct((2, B, HH), jnp.float32),
        mesh=mesh,
        scratch_types=common_scratch + [
            pltpu.VMEM((TPS,), jnp.int32),
            pltpu.VMEM((TPS // 4, HH), jnp.float32),
        ],
        compiler_params=_SC_PARAMS,
    )


_RB = 256


def _split_store(z3_ref, z):
    z3_ref[0] = z[:, :HH]
    z3_ref[1] = z[:, HH:]


def _tc_in_body(deg_ref, x_ref, w_ref, z3_ref):
    dinv = lax.rsqrt(deg_ref[...] + 1.0)
    z = jnp.dot(x_ref[...], w_ref[...],
                preferred_element_type=jnp.float32) * dinv
    _split_store(z3_ref, z)


def _make_tc_in():
    grid = (NPAD // _RB,)
    return pl.pallas_call(
        _tc_in_body,
        grid=grid,
        in_specs=[
            pl.BlockSpec((_RB, 1), lambda i: (i, 0)),
            pl.BlockSpec((_RB, D), lambda i: (i, 0)),
            pl.BlockSpec((D, H), lambda i: (0, 0)),
        ],
        out_specs=pl.BlockSpec((2, _RB, HH), lambda i: (0, i, 0)),
        out_shape=jax.ShapeDtypeStruct((2, NPAD, HH), jnp.float32),
    )


def _tc_mid_body(acc3_ref, deg_ref, b_ref, w_ref, z3_ref):
    acc = jnp.concatenate([acc3_ref[0], acc3_ref[1]], axis=1)
    dinv = lax.rsqrt(deg_ref[...] + 1.0)
    h = jnp.maximum(acc * dinv + b_ref[...], 0.0)
    z = jnp.dot(h, w_ref[...], preferred_element_type=jnp.float32) * dinv
    _split_store(z3_ref, z)


def _make_tc_mid():
    grid = (NPAD // _RB,)
    return pl.pallas_call(
        _tc_mid_body,
        grid=grid,
        in_specs=[
            pl.BlockSpec((2, _RB, HH), lambda i: (0, i, 0)),
            pl.BlockSpec((_RB, 1), lambda i: (i, 0)),
            pl.BlockSpec((1, H), lambda i: (0, 0)),
            pl.BlockSpec((H, H), lambda i: (0, 0)),
        ],
        out_specs=pl.BlockSpec((2, _RB, HH), lambda i: (0, i, 0)),
        out_shape=jax.ShapeDtypeStruct((2, NPAD, HH), jnp.float32),
    )


def _gates(g):
    i = jax.nn.sigmoid(g[:, 0:H])
    gg = jnp.tanh(g[:, 2 * H:3 * H])
    o = jax.nn.sigmoid(g[:, 3 * H:4 * H])
    return o * jnp.tanh(i * gg)


def _tc_head_body(t3_ref, deg_ref, ti_ref, b2_ref, wih0_ref, b0_ref,
                  wih1_ref, b1_ref, fcw_ref, fcb_ref, o_ref):
    dn = (((1,), (1,)), ((), ()))
    traw = jnp.concatenate([t3_ref[0], t3_ref[1]], axis=1)
    tid = ti_ref[...]
    KC = 2048

    def chunk(k, acc):
        kb = k * KC
        ids = kb + lax.broadcasted_iota(jnp.int32, (1, KC), 1)
        oh = (tid == ids).astype(jnp.float32)
        dsl = deg_ref[pl.ds(kb, KC), 0:1]
        return acc + jnp.dot(oh, dsl, preferred_element_type=jnp.float32)

    degt = lax.fori_loop(0, NPAD // KC, chunk,
                         jnp.zeros((_RB, 1), jnp.float32))
    dinv_t = lax.rsqrt(degt + 1.0)
    t = jnp.maximum(traw * dinv_t + b2_ref[...], 0.0)
    g0 = lax.dot_general(t, wih0_ref[...], dn,
                         preferred_element_type=jnp.float32) + b0_ref[...]
    h1 = _gates(g0)
    g1 = lax.dot_general(h1, wih1_ref[...], dn,
                         preferred_element_type=jnp.float32) + b1_ref[...]
    h2 = _gates(g1)
    o_ref[...] = lax.dot_general(h2, fcw_ref[...], dn,
                                 preferred_element_type=jnp.float32) + fcb_ref[...]


def _make_tc_head():
    grid = (B // _RB,)
    return pl.pallas_call(
        _tc_head_body,
        grid=grid,
        in_specs=[
            pl.BlockSpec((2, _RB, HH), lambda i: (0, i, 0)),
            pl.BlockSpec((NPAD, 1), lambda i: (0, 0)),
            pl.BlockSpec((_RB, 1), lambda i: (i, 0)),
            pl.BlockSpec((1, H), lambda i: (0, 0)),
            pl.BlockSpec((4 * H, H), lambda i: (0, 0)),
            pl.BlockSpec((1, 4 * H), lambda i: (0, 0)),
            pl.BlockSpec((4 * H, H), lambda i: (0, 0)),
            pl.BlockSpec((1, 4 * H), lambda i: (0, 0)),
            pl.BlockSpec((NCLS, H), lambda i: (0, 0)),
            pl.BlockSpec((1, NCLS), lambda i: (0, 0)),
        ],
        out_specs=pl.BlockSpec((_RB, NCLS), lambda i: (i, 0)),
        out_shape=jax.ShapeDtypeStruct((B, NCLS), jnp.float32),
    )


def kernel(x, edge_index, target_node_index, W1, b1, W2, b2, Wih0, Whh0,
           bih0, bhh0, Wih1, Whh1, bih1, bhh1, fcW, fcb):
    src = edge_index[0]
    dst = edge_index[1]
    ppe = EPSP - EPS
    padv = (N + jnp.arange(16 * ppe).reshape(16, ppe) % (NPAD - N)
            ).astype(src.dtype)
    srcp = jnp.concatenate([src.reshape(16, EPS), padv], axis=1).reshape(-1)
    dstp = jnp.concatenate([dst.reshape(16, EPS), padv], axis=1).reshape(-1)
    x_pad = jnp.pad(x, ((0, NPAD - N), (0, 0)))

    deg = _make_sc_degree()(dst).reshape(NPAD, 1)

    z3 = _make_tc_in()(deg, x_pad, W1)
    acc3 = _make_sc_spmm("acc")(z3, srcp, dstp)
    z3 = _make_tc_mid()(acc3, deg, b1.reshape(1, H), W2)
    t3 = _make_sc_spmm("tgt")(z3, srcp, dstp, target_node_index)
    return _make_tc_head()(t3, deg,
                           target_node_index.reshape(B, 1),
                           b2.reshape(1, H),
                           Wih0, (bih0 + bhh0).reshape(1, 4 * H),
                           Wih1, (bih1 + bhh1).reshape(1, 4 * H),
                           fcW, fcb.reshape(1, NCLS))


# speedup vs baseline: 1.0553x; 1.0348x over previous
import jax
import jax.numpy as jnp
from jax import lax
from jax.experimental import pallas as pl
from jax.experimental.pallas import tpu as pltpu
from jax.experimental.pallas import tpu_sc as plsc

N = 10000
NPAD = 10240
OWN = NPAD // 2
TRASH = OWN
E = 160000
EPS = E // 16
BLK_ACC = 64
EPSP = 10048
BLK_TGT = 80
D = 256
H = 256
HH = 128
NCLS = 64
B = 1024

_SC_MESH = dict(core_axis_name="c", subcore_axis_name="s")
_SC_PARAMS = pltpu.CompilerParams(needs_layout_passes=False)


HSIZE = 5376


def _make_sc_degree():
    mesh = plsc.VectorSubcoreMesh(**_SC_MESH)

    def body(dst_hbm, deg_hbm, dstb, hist, rbuf, sbuf, sh, sem):
        c = lax.axis_index("c")
        s = lax.axis_index("s")

        def zero_chunk(i, _):
            hist[pl.ds(i * 16, 16)] = jnp.zeros((16,), jnp.float32)
            return _
        lax.fori_loop(0, HSIZE // 16, zero_chunk, None)

        pltpu.sync_copy(dst_hbm.at[pl.ds(s * EPS, EPS)], dstb)

        cbase = c * OWN

        def count(i, _):
            d = dstb[pl.ds(i * 16, 16)]
            dl = d - cbase
            ok = (dl >= 0) & (dl < OWN)
            idx = jnp.where(ok, dl, TRASH)
            plsc.addupdate_scatter(hist, [idx], jnp.ones((16,), jnp.float32))
            return _
        lax.fori_loop(0, EPS // 16, count, None)

        pltpu.sync_copy(hist, sh.at[s])
        plsc.subcore_barrier()

        @pl.when(s < 8)
        def _():
            for k in range(16):
                pltpu.sync_copy(sh.at[k].at[pl.ds(s * 640, 640)], rbuf.at[k])

            def red_chunk(i, _):
                acc = rbuf[0, pl.ds(i * 16, 16)]
                for k in range(1, 16):
                    acc = acc + rbuf[k, pl.ds(i * 16, 16)]
                sbuf[pl.ds(i * 16, 16)] = acc
                return _
            lax.fori_loop(0, 40, red_chunk, None)

            pltpu.sync_copy(sbuf, deg_hbm.at[pl.ds(cbase + s * 640, 640)])

    return pl.kernel(
        body,
        out_type=jax.ShapeDtypeStruct((NPAD,), jnp.float32),
        mesh=mesh,
        scratch_types=[
            pltpu.VMEM((EPS,), jnp.int32),
            pltpu.VMEM((HSIZE,), jnp.float32),
            pltpu.VMEM((16, 640), jnp.float32),
            pltpu.VMEM((640,), jnp.float32),
            pltpu.VMEM_SHARED((16, HSIZE), jnp.float32),
            pltpu.SemaphoreType.DMA,
        ],
        compiler_params=_SC_PARAMS,
    )


def _make_sc_spmm(tail):
    mesh = plsc.VectorSubcoreMesh(**_SC_MESH)

    NS = 3
    blk = BLK_ACC
    eps = EPSP
    nblk = eps // blk

    def spmm_body(z_hbm, src_hbm, dst_hbm, srcb, dstb, srcw, idxw, gbuf,
                  acc_sh, gsem, ssem):
        c = lax.axis_index("c")
        s = lax.axis_index("s")
        zc = z_hbm.at[c]

        pltpu.sync_copy(zc.at[pl.ds(s * 640, 640)],
                        acc_sh.at[pl.ds(s * 640, 640)])

        pltpu.sync_copy(src_hbm.at[pl.ds(s * eps, eps)], srcb)
        pltpu.sync_copy(dst_hbm.at[pl.ds(s * eps, eps)], dstb)

        plsc.subcore_barrier()

        def stage_and_fire(b, k):
            base = b * blk
            for j in range(blk // 16):
                idxw[k, pl.ds(j * 16, 16)] = dstb[pl.ds(base + j * 16, 16)]
                srcw[k, pl.ds(j * 16, 16)] = srcb[pl.ds(base + j * 16, 16)]
            pltpu.async_copy(zc.at[srcw.at[k]], gbuf.at[k], gsem[k])

        def finish(k):
            pltpu.make_async_copy(zc.at[srcw.at[k]], gbuf.at[k],
                                  gsem[k]).wait()
            pltpu.async_copy(gbuf.at[k], acc_sh.at[idxw.at[k]], ssem[k],
                             add=True)

        def drain(k):
            pltpu.make_async_copy(gbuf.at[k], acc_sh.at[idxw.at[k]],
                                  ssem[k]).wait()

        stage_and_fire(0, 0)

        def group(g, _):
            for k in range(NS):
                b = g * NS + k
                nk = (k + 1) % NS
                if k == NS - 1:
                    drain(nk)
                else:
                    @pl.when(g > 0)
                    def _():
                        drain(nk)
                stage_and_fire(b + 1, nk)
                finish(k)
            return _
        lax.fori_loop(0, (nblk - 1) // NS, group, None)
        finish(0)
        for k in range(1, NS):
            drain(k)
        drain(0)

        plsc.subcore_barrier()
        return c, s

    common_scratch = [
        pltpu.VMEM((eps,), jnp.int32),
        pltpu.VMEM((eps,), jnp.int32),
        pltpu.VMEM((NS, blk), jnp.int32),
        pltpu.VMEM((NS, blk), jnp.int32),
        pltpu.VMEM((NS, blk, HH), jnp.float32),
        pltpu.VMEM_SHARED((NPAD, HH), jnp.float32),
        [pltpu.SemaphoreType.DMA] * NS,
        [pltpu.SemaphoreType.DMA] * NS,
    ]

    if tail == "acc":
        def body(z_hbm, src_hbm, dst_hbm, acc_hbm, srcb, dstb, srcw, idxw,
                 gbuf, acc_sh, gsem, ssem):
            c, s = spmm_body(z_hbm, src_hbm, dst_hbm, srcb, dstb, srcw, idxw,
                             gbuf, acc_sh, gsem, ssem)
            pltpu.sync_copy(acc_sh.at[pl.ds(s * 640, 640)],
                            acc_hbm.at[c].at[pl.ds(s * 640, 640)])

        return pl.kernel(
            body,
            out_type=jax.ShapeDtypeStruct((2, NPAD, HH), jnp.float32),
            mesh=mesh,
            scratch_types=common_scratch,
            compiler_params=_SC_PARAMS,
        )

    TPS = B // 16

    def body(z_hbm, src_hbm, dst_hbm, tgt_hbm, t3_hbm,
             srcb, dstb, srcw, idxw, gbuf, acc_sh, gsem, ssem, tbuf, trows):
        c, s = spmm_body(z_hbm, src_hbm, dst_hbm, srcb, dstb, srcw, idxw,
                         gbuf, acc_sh, gsem, ssem)
        base = s * TPS
        pltpu.sync_copy(tgt_hbm.at[pl.ds(base, TPS)], tbuf)
        def tgt_round(h, _):
            off = h * (TPS // 4)
            tb = tbuf.at[pl.ds(off, TPS // 4)]
            pltpu.async_copy(acc_sh.at[tb], trows, gsem[0]).wait()
            pltpu.sync_copy(
                trows, t3_hbm.at[c].at[pl.ds(base + off, TPS // 4)])
            return _
        lax.fori_loop(0, 4, tgt_round, None)

    return pl.kernel(
        body,
        out_type=jax.ShapeDtypeStruct((2, B, HH), jnp.float32),
        mesh=mesh,
        scratch_types=common_scratch + [
            pltpu.VMEM((TPS,), jnp.int32),
            pltpu.VMEM((TPS // 4, HH), jnp.float32),
        ],
        compiler_params=_SC_PARAMS,
    )


_RB = 256


def _split_store(z3_ref, z):
    z3_ref[0] = z[:, :HH]
    z3_ref[1] = z[:, HH:]


def _tc_in_body(deg_ref, x_ref, w_ref, z3_ref):
    dinv = lax.rsqrt(deg_ref[...] + 1.0)
    z = jnp.dot(x_ref[...], w_ref[...],
                preferred_element_type=jnp.float32) * dinv
    _split_store(z3_ref, z)


def _make_tc_in():
    grid = (NPAD // _RB,)
    return pl.pallas_call(
        _tc_in_body,
        grid=grid,
        in_specs=[
            pl.BlockSpec((_RB, 1), lambda i: (i, 0)),
            pl.BlockSpec((_RB, D), lambda i: (i, 0)),
            pl.BlockSpec((D, H), lambda i: (0, 0)),
        ],
        out_specs=pl.BlockSpec((2, _RB, HH), lambda i: (0, i, 0)),
        out_shape=jax.ShapeDtypeStruct((2, NPAD, HH), jnp.float32),
    )


def _tc_mid_body(acc3_ref, deg_ref, b_ref, w_ref, z3_ref):
    acc = jnp.concatenate([acc3_ref[0], acc3_ref[1]], axis=1)
    dinv = lax.rsqrt(deg_ref[...] + 1.0)
    h = jnp.maximum(acc * dinv + b_ref[...], 0.0)
    z = jnp.dot(h, w_ref[...], preferred_element_type=jnp.float32) * dinv
    _split_store(z3_ref, z)


def _make_tc_mid():
    grid = (NPAD // _RB,)
    return pl.pallas_call(
        _tc_mid_body,
        grid=grid,
        in_specs=[
            pl.BlockSpec((2, _RB, HH), lambda i: (0, i, 0)),
            pl.BlockSpec((_RB, 1), lambda i: (i, 0)),
            pl.BlockSpec((1, H), lambda i: (0, 0)),
            pl.BlockSpec((H, H), lambda i: (0, 0)),
        ],
        out_specs=pl.BlockSpec((2, _RB, HH), lambda i: (0, i, 0)),
        out_shape=jax.ShapeDtypeStruct((2, NPAD, HH), jnp.float32),
    )


def _gates(g):
    i = jax.nn.sigmoid(g[:, 0:H])
    gg = jnp.tanh(g[:, 2 * H:3 * H])
    o = jax.nn.sigmoid(g[:, 3 * H:4 * H])
    return o * jnp.tanh(i * gg)


def _tc_head_body(t3_ref, deg_ref, ti_ref, b2_ref, wih0_ref, b0_ref,
                  wih1_ref, b1_ref, fcw_ref, fcb_ref, o_ref):
    dn = (((1,), (1,)), ((), ()))
    traw = jnp.concatenate([t3_ref[0], t3_ref[1]], axis=1)
    tid = ti_ref[...]
    KC = 2048

    def chunk(k, acc):
        kb = k * KC
        ids = kb + lax.broadcasted_iota(jnp.int32, (1, KC), 1)
        oh = (tid == ids).astype(jnp.float32)
        dsl = deg_ref[pl.ds(kb, KC), 0:1]
        return acc + jnp.dot(oh, dsl, preferred_element_type=jnp.float32)

    degt = lax.fori_loop(0, NPAD // KC, chunk,
                         jnp.zeros((_RB, 1), jnp.float32))
    dinv_t = lax.rsqrt(degt + 1.0)
    t = jnp.maximum(traw * dinv_t + b2_ref[...], 0.0)
    g0 = lax.dot_general(t, wih0_ref[...], dn,
                         preferred_element_type=jnp.float32) + b0_ref[...]
    h1 = _gates(g0)
    g1 = lax.dot_general(h1, wih1_ref[...], dn,
                         preferred_element_type=jnp.float32) + b1_ref[...]
    h2 = _gates(g1)
    o_ref[...] = lax.dot_general(h2, fcw_ref[...], dn,
                                 preferred_element_type=jnp.float32) + fcb_ref[...]


def _make_tc_head():
    grid = (B // _RB,)
    return pl.pallas_call(
        _tc_head_body,
        grid=grid,
        in_specs=[
            pl.BlockSpec((2, _RB, HH), lambda i: (0, i, 0)),
            pl.BlockSpec((NPAD, 1), lambda i: (0, 0)),
            pl.BlockSpec((_RB, 1), lambda i: (i, 0)),
            pl.BlockSpec((1, H), lambda i: (0, 0)),
            pl.BlockSpec((4 * H, H), lambda i: (0, 0)),
            pl.BlockSpec((1, 4 * H), lambda i: (0, 0)),
            pl.BlockSpec((4 * H, H), lambda i: (0, 0)),
            pl.BlockSpec((1, 4 * H), lambda i: (0, 0)),
            pl.BlockSpec((NCLS, H), lambda i: (0, 0)),
            pl.BlockSpec((1, NCLS), lambda i: (0, 0)),
        ],
        out_specs=pl.BlockSpec((_RB, NCLS), lambda i: (i, 0)),
        out_shape=jax.ShapeDtypeStruct((B, NCLS), jnp.float32),
    )


def kernel(x, edge_index, target_node_index, W1, b1, W2, b2, Wih0, Whh0,
           bih0, bhh0, Wih1, Whh1, bih1, bhh1, fcW, fcb):
    src = edge_index[0]
    dst = edge_index[1]
    ppe = EPSP - EPS
    padv = (N + jnp.arange(16 * ppe).reshape(16, ppe) % (NPAD - N)
            ).astype(src.dtype)
    srcp = jnp.concatenate([src.reshape(16, EPS), padv], axis=1).reshape(-1)
    dstp = jnp.concatenate([dst.reshape(16, EPS), padv], axis=1).reshape(-1)
    x_pad = jnp.pad(x, ((0, NPAD - N), (0, 0)))

    deg = _make_sc_degree()(dst).reshape(NPAD, 1)

    z3 = _make_tc_in()(deg, x_pad, W1)
    acc3 = _make_sc_spmm("acc")(z3, srcp, dstp)
    z3 = _make_tc_mid()(acc3, deg, b1.reshape(1, H), W2)
    t3 = _make_sc_spmm("tgt")(z3, srcp, dstp, target_node_index)
    return _make_tc_head()(t3, deg,
                           target_node_index.reshape(B, 1),
                           b2.reshape(1, H),
                           Wih0, (bih0 + bhh0).reshape(1, 4 * H),
                           Wih1, (bih1 + bhh1).reshape(1, 4 * H),
                           fcW, fcb.reshape(1, NCLS))
